# baseline (speedup 1.0000x reference)
import jax
import jax.numpy as jnp
from jax import lax
from jax.experimental import pallas as pl
from jax.experimental.pallas import tpu as pltpu
from jax.experimental.pallas import tpu_sc as plsc

M_ITEM = 1_000_000
DIM = 32
B = 16384
LR = 0.01
GRAD_LIMIT = 5.0

NC = 2
NS = 16
L = 16
PP = B // NS
WPC = 15
NWIN = NC * WPC
CAP = 33334
WCAP = PP + L


def _rsqrt(x):
    i = plsc.bitcast(x, jnp.int32)
    y = plsc.bitcast(jnp.int32(0x5F3759DF) - (i >> 1), jnp.float32)
    for _ in range(4):
        y = y * (jnp.float32(1.5) - jnp.float32(0.5) * x * y * y)
    return y


def _smax(v):
    return lax.reduce_max(v, axes=(0,))


def _sc_body(tbl, grad, items, idxb, wpk, cbase, gbuf, ctmp, zbuf, cnts_s, acc):
    cid = lax.axis_index("c")
    sid = lax.axis_index("s")
    iota = lax.iota(jnp.int32, L)

    pltpu.sync_copy(items.at[pl.ds(sid * PP, PP)], idxb)
    zf = jnp.zeros((L,), jnp.float32)
    for r in range(L):
        zbuf[r, pl.ds(0, L)] = zf
        zbuf[r, pl.ds(L, L)] = zf
    for w in range(WPC):
        cnts_s[w] = jnp.int32(0)

    def part_body(k, c):
        idxv = idxb[pl.ds(k * L, L)]
        wv = idxv // jnp.int32(CAP)
        pkv = (idxv << 10) | (k * L + iota)
        for w in range(WPC):
            m = wv == (cid * WPC + w)
            cnt = cnts_s[w]
            plsc.store_compressed(wpk.at[pl.ds(w * WCAP + cnt, L)], pkv, mask=m)
            cnts_s[w] = cnt + _smax(plsc.all_reduce_population_count(m))
        return c

    lax.fori_loop(0, PP // L, part_body, 0)

    def win_body(w, carry):
        n = cnts_s[w]
        nch = (n + (L - 1)) // L
        base = (cid * WPC + w) * CAP
        lbase = w * WCAP

        def win_idx(j, n_=n, lbase_=lbase):
            pkv = wpk[pl.ds(lbase_ + j * L, L)]
            m = iota < (n_ - j * L)
            safe = _smax(jnp.where(m, pkv, jnp.int32(0)))
            pkv = jnp.where(m, pkv, safe)
            return pkv >> 10, pkv & jnp.int32(1023), m

        def zero_body(j, c, n_=n, lbase_=lbase, base_=base):
            idxv, _, _ = win_idx(j, n_, lbase_)
            pltpu.sync_copy(zbuf, acc.at[idxv - base_])
            return c

        def acc_body(j, c, n_=n, lbase_=lbase, base_=base):
            idxv, lposv, m = win_idx(j, n_, lbase_)
            pltpu.sync_copy(grad.at[sid * PP + lposv], gbuf)
            ssq = jnp.zeros((L,), jnp.float32)
            for col in range(DIM):
                cv = jnp.full((L,), col, jnp.int32)
                v = plsc.load_gather(gbuf, [iota, cv])
                ssq = ssq + v * v
            lim2 = jnp.float32(GRAD_LIMIT * GRAD_LIMIT)
            scale = jnp.where(ssq > lim2,
                              jnp.float32(GRAD_LIMIT) * _rsqrt(ssq),
                              jnp.float32(1.0)) * jnp.float32(-LR)
            for col in range(DIM):
                cv = jnp.full((L,), col, jnp.int32)
                v = plsc.load_gather(gbuf, [iota, cv]) * scale
                v = jnp.where(m, v, jnp.float32(0.0))
                plsc.store_scatter(gbuf, [iota, cv], v)
            pltpu.sync_copy(gbuf, acc.at[idxv - base_], add=True)
            return c

        def gather_body(j, c, n_=n, lbase_=lbase, base_=base):
            idxv, _, _ = win_idx(j, n_, lbase_)
            pltpu.sync_copy(tbl.at[idxv], cbase.at[pl.ds(j * L, L)])
            pltpu.sync_copy(acc.at[idxv - base_], ctmp)
            for r in range(L):
                for h in range(2):
                    sl = pl.ds(h * L, L)
                    cbase[j * L + r, sl] = cbase[j * L + r, sl] + ctmp[r, sl]
            return c

        def scat_body(j, c, n_=n, lbase_=lbase):
            idxv, _, _ = win_idx(j, n_, lbase_)
            pltpu.sync_copy(cbase.at[pl.ds(j * L, L)], tbl.at[idxv])
            return c

        lax.fori_loop(0, nch, zero_body, 0)
        plsc.subcore_barrier()
        lax.fori_loop(0, nch, acc_body, 0)
        plsc.subcore_barrier()
        lax.fori_loop(0, nch, gather_body, 0)
        plsc.subcore_barrier()
        lax.fori_loop(0, nch, scat_body, 0)
        return carry

    lax.fori_loop(0, WPC, win_body, 0)


def _make_sc():
    mesh = plsc.VectorSubcoreMesh(
        core_axis_name="c", subcore_axis_name="s",
        num_cores=NC, num_subcores=NS)
    return pl.kernel(
        _sc_body,
        out_type=(),
        mesh=mesh,
        compiler_params=pltpu.CompilerParams(
            needs_layout_passes=False, use_tc_tiling_on_sc=False),
        scratch_types=[
            pltpu.VMEM((PP,), jnp.int32),
            pltpu.VMEM((WPC * WCAP,), jnp.int32),
            pltpu.VMEM((PP, DIM), jnp.float32),
            pltpu.VMEM((L, DIM), jnp.float32),
            pltpu.VMEM((L, DIM), jnp.float32),
            pltpu.VMEM((L, DIM), jnp.float32),
            pltpu.SMEM((WPC,), jnp.int32),
            pltpu.VMEM_SHARED((CAP, DIM), jnp.float32),
        ],
    )


def kernel(items_emb, items_emb_grad, items):
    tbl = jax.new_ref(items_emb)
    _make_sc()(tbl, items_emb_grad, items)
    return jax.freeze(tbl)
